# Initial kernel scaffold; baseline (speedup 1.0000x reference)
import jax
import jax.numpy as jnp
from jax.experimental import pallas as pl


def kernel(scores):
    raise NotImplementedError("write your pallas kernel here")


# speedup vs baseline: 6.3092x; 6.3092x over previous
import functools

import jax
import jax.numpy as jnp
from jax.experimental import pallas as pl
from jax.experimental.pallas import tpu as pltpu

_B = 4096
_K1 = 21
_ROWS = 64
_LOG2 = 0.6931471805599453


def _neg_log_sigmoid(d):
    return jnp.maximum(-d, 0.0) + jnp.log1p(jnp.exp(-jnp.abs(d)))


def _topk_body(x_ref, out_ref, s_ref, acc_ref):
    i = pl.program_id(0)
    nsteps = pl.num_programs(0)

    x0 = x_ref[...]
    cols = jax.lax.broadcasted_iota(jnp.int32, (_ROWS, _B), 1)
    rowg = i * _ROWS + jax.lax.broadcasted_iota(jnp.int32, (_ROWS, _B), 0)
    neginf = jnp.float32(-jnp.inf)

    is_diag = cols == rowg
    pos = jnp.max(jnp.where(is_diag, x0, neginf), axis=1, keepdims=True)

    cnt_gt = jnp.sum((x0 > pos).astype(jnp.float32), axis=1, keepdims=True)
    cnt_eq = jnp.sum(((x0 == pos) & (cols < rowg)).astype(jnp.float32),
                     axis=1, keepdims=True)
    rank_self = cnt_gt + cnt_eq
    self_in = rank_self < _K1

    sarange = jax.lax.broadcasted_iota(jnp.int32, (1, _K1), 1).astype(jnp.float32)
    wrow = 1.0 / jnp.log2(sarange + 2.0)
    w_self = jnp.sum(jnp.where(sarange == rank_self, wrow, 0.0),
                     axis=1, keepdims=True)
    loss_self = jnp.where(self_in, w_self * _LOG2, 0.0)
    mask_rows = jnp.where(self_in, _K1 - 1.0, float(_K1))

    s_ref[...] = x0

    def body(_, carry):
        r_cur, loss_row = carry
        xm = s_ref[...]
        m = jnp.max(xm, axis=1, keepdims=True)
        eq = xm == m
        c = jnp.sum(eq.astype(jnp.float32), axis=1, keepdims=True)
        s_ref[...] = jnp.where(eq, neginf, xm)
        in_win = (sarange >= r_cur) & (sarange < r_cur + c)
        wsum = jnp.sum(jnp.where(in_win, wrow, 0.0), axis=1, keepdims=True)
        f = _neg_log_sigmoid(pos - m)
        loss_row = loss_row + jnp.where(wsum > 0.0, f * wsum, 0.0)
        return r_cur + c, loss_row

    zero = jnp.zeros((_ROWS, 1), jnp.float32)
    _, loss_row = jax.lax.fori_loop(0, _K1, body, (zero, zero))

    blk_loss = jnp.sum(loss_row - loss_self)
    blk_mask = jnp.sum(mask_rows)

    @pl.when(i == 0)
    def _():
        acc_ref[0] = 0.0
        acc_ref[1] = 0.0

    acc_ref[0] += blk_loss
    acc_ref[1] += blk_mask
    out_ref[...] = jnp.full((1, 1), acc_ref[0] / jnp.maximum(acc_ref[1], 1.0),
                            jnp.float32)


@jax.jit
def kernel(scores):
    nsteps = _B // _ROWS
    out = pl.pallas_call(
        _topk_body,
        grid=(nsteps,),
        in_specs=[pl.BlockSpec((_ROWS, _B), lambda i: (i, 0))],
        out_specs=pl.BlockSpec((1, 1), lambda i: (0, 0)),
        out_shape=jax.ShapeDtypeStruct((1, 1), jnp.float32),
        scratch_shapes=[
            pltpu.VMEM((_ROWS, _B), jnp.float32),
            pltpu.SMEM((2,), jnp.float32),
        ],
    )(scores)
    return jnp.reshape(out, ())
